# baseline (speedup 1.0000x reference)
import functools

import jax
import jax.numpy as jnp
from jax import lax
from jax.experimental import pallas as pl
from jax.experimental.pallas import tpu as pltpu
from jax.experimental.pallas import tpu_sc as plsc

N = 10000
E = 320000
D = 128
DE = 4

NC = 2
NS = 16
NW = NC * NS
EPW = E // NW
CH = 80
NCH = EPW // CH
AW = D + 16
RPT = N // NS

_PREC = jax.lax.Precision.HIGHEST


def _hext_store(h_ref, av_ref, h, asrc_vec, adst_vec):
    h_ref[:, :D] = h
    h_ref[:, D:] = jnp.dot(h, asrc_vec, precision=_PREC,
                           preferred_element_type=jnp.float32)
    av_ref[...] = jnp.dot(h, adst_vec, precision=_PREC,
                          preferred_element_type=jnp.float32)


def _tc_pre(x, W, As, Ad):
    R = 2000

    def body(x_ref, w_ref, as_ref, ad_ref, h_ref, av_ref):
        h = jnp.dot(x_ref[...], w_ref[...], precision=_PREC,
                    preferred_element_type=jnp.float32)
        _hext_store(h_ref, av_ref, h, as_ref[...], ad_ref[...])

    return pl.pallas_call(
        body,
        grid=(N // R,),
        in_specs=[
            pl.BlockSpec((R, D), lambda i: (i, 0)),
            pl.BlockSpec((D, D), lambda i: (0, 0)),
            pl.BlockSpec((D, 16), lambda i: (0, 0)),
            pl.BlockSpec((D, 8), lambda i: (0, 0)),
        ],
        out_specs=[
            pl.BlockSpec((R, AW), lambda i: (i, 0)),
            pl.BlockSpec((R, 8), lambda i: (i, 0)),
        ],
        out_shape=[
            jax.ShapeDtypeStruct((N, AW), jnp.float32),
            jax.ShapeDtypeStruct((N, 8), jnp.float32),
        ],
    )(x, W, As, Ad)


def _tc_mid(acc, b1, W2, As, Ad):
    R = 2000

    def body(acc_ref, b_ref, w_ref, as_ref, ad_ref, h_ref, av_ref):
        num = acc_ref[0] + acc_ref[1]
        den = num[:, D:D + 1] + 1e-16
        x2 = jnp.maximum(num[:, :D] / den + b_ref[...], 0.0)
        h2 = jnp.dot(x2, w_ref[...], precision=_PREC,
                     preferred_element_type=jnp.float32)
        _hext_store(h_ref, av_ref, h2, as_ref[...], ad_ref[...])

    return pl.pallas_call(
        body,
        grid=(N // R,),
        in_specs=[
            pl.BlockSpec((2, R, AW), lambda i: (0, i, 0)),
            pl.BlockSpec((1, D), lambda i: (0, 0)),
            pl.BlockSpec((D, D), lambda i: (0, 0)),
            pl.BlockSpec((D, 16), lambda i: (0, 0)),
            pl.BlockSpec((D, 8), lambda i: (0, 0)),
        ],
        out_specs=[
            pl.BlockSpec((R, AW), lambda i: (i, 0)),
            pl.BlockSpec((R, 8), lambda i: (i, 0)),
        ],
        out_shape=[
            jax.ShapeDtypeStruct((N, AW), jnp.float32),
            jax.ShapeDtypeStruct((N, 8), jnp.float32),
        ],
    )(acc, b1, W2, As, Ad)


def _tc_final(acc, b2):
    R = 2000

    def body(acc_ref, b_ref, o_ref):
        num = acc_ref[0] + acc_ref[1]
        den = num[:, D:D + 1] + 1e-16
        o_ref[...] = num[:, :D] / den + b_ref[...]

    return pl.pallas_call(
        body,
        grid=(N // R,),
        in_specs=[
            pl.BlockSpec((2, R, AW), lambda i: (0, i, 0)),
            pl.BlockSpec((1, D), lambda i: (0, 0)),
        ],
        out_specs=pl.BlockSpec((R, D), lambda i: (i, 0)),
        out_shape=jax.ShapeDtypeStruct((N, D), jnp.float32),
    )(acc, b2)


SB = 5
NSUP = NCH // SB
SCW = SB * CH


def _sc_layer(h_ext, adst, we, eif, ea4):
    mesh = plsc.VectorSubcoreMesh(core_axis_name="c", subcore_axis_name="s",
                                  num_cores=NC, num_subcores=NS)

    @functools.partial(
        pl.kernel,
        out_type=jax.ShapeDtypeStruct((NC, N, AW), jnp.float32),
        mesh=mesh,
        compiler_params=pltpu.CompilerParams(use_tc_tiling_on_sc=False,
                                             needs_layout_passes=False),
        scratch_types=[
            pltpu.VMEM((N,), jnp.float32),
            pltpu.VMEM((16,), jnp.float32),
            pltpu.VMEM((2, SCW), jnp.int32),
            pltpu.VMEM((2, SCW), jnp.int32),
            pltpu.VMEM((2, SCW * DE), jnp.float32),
            pltpu.VMEM((2, CH), jnp.int32),
            pltpu.VMEM((2, CH), jnp.int32),
            pltpu.VMEM((2, CH, AW), jnp.float32),
            pltpu.VMEM((CH,), jnp.float32),
            pltpu.VMEM_SHARED((N, AW), jnp.float32),
            pltpu.SemaphoreType.DMA,
            pltpu.SemaphoreType.DMA,
            pltpu.SemaphoreType.DMA,
            pltpu.SemaphoreType.DMA,
            pltpu.SemaphoreType.DMA,
        ],
    )
    def k(h_hbm, adst_hbm, we_hbm, ei_hbm, ea_hbm, acc_hbm,
          adst_v, we_v, ssrc_v, sdst_v, sea_v, sidx_v, didx_v, rows_v,
          exbuf_v, acc_s, semg0, semg1, sempf, semsc0, semsc1):
        cid = lax.axis_index("c")
        sid = lax.axis_index("s")
        wid = sid * NC + cid
        semg = (semg0, semg1)
        semsc = (semsc0, semsc1)

        pltpu.sync_copy(adst_hbm, adst_v)
        pltpu.sync_copy(we_hbm, we_v)

        zero16 = jnp.zeros((16,), jnp.float32)

        @pl.loop(0, CH)
        def _(r):
            for c in range(AW // 16):
                rows_v[0, r, pl.ds(c * 16, 16)] = zero16

        for kk in range(RPT // CH):
            pltpu.sync_copy(rows_v.at[0],
                            acc_s.at[pl.ds(sid * RPT + kk * CH, CH)])
        rem = RPT % CH
        if rem:
            pltpu.sync_copy(rows_v.at[0, pl.ds(0, rem)],
                            acc_s.at[pl.ds(sid * RPT + RPT - rem, rem)])
        plsc.subcore_barrier()

        lane = lax.iota(jnp.int32, 16)
        wv = we_v[...]
        wb = [jnp.broadcast_to(wv[j], (16,)) for j in range(DE)]

        def pf_copies(s, p):
            eoff = wid * EPW + s * SCW
            return (
                pltpu.make_async_copy(ei_hbm.at[pl.ds(eoff, SCW)],
                                      ssrc_v.at[p], sempf),
                pltpu.make_async_copy(ei_hbm.at[pl.ds(E + eoff, SCW)],
                                      sdst_v.at[p], sempf),
                pltpu.make_async_copy(ea_hbm.at[pl.ds(eoff * DE, SCW * DE)],
                                      sea_v.at[p], sempf),
            )

        def copy_idx(p, j, slot):
            for g in range(CH // 16):
                sidx_v[slot, pl.ds(g * 16, 16)] = (
                    ssrc_v[p, pl.ds(j * CH + g * 16, 16)])
                didx_v[slot, pl.ds(g * 16, 16)] = (
                    sdst_v[p, pl.ds(j * CH + g * 16, 16)])

        def compute_ex(p, j, q):
            colD = jnp.full((16,), D, jnp.int32)
            qf = jnp.full((16,), q, jnp.int32)
            pf = jnp.broadcast_to(p, (16,))
            for g in range(CH // 16):
                rowi = g * 16 + lane
                a = (plsc.load_gather(rows_v, [qf, rowi, colD])
                     + plsc.load_gather(adst_v,
                                        [didx_v[q, pl.ds(g * 16, 16)]]))
                eb = (j * CH + g * 16 + lane) * DE
                ae = wb[0] * plsc.load_gather(sea_v, [pf, eb])
                for jj in range(1, DE):
                    ae = ae + wb[jj] * plsc.load_gather(sea_v, [pf, eb + jj])
                al = a + ae
                al = jnp.maximum(al, 0.2 * al)
                exbuf_v[pl.ds(g * 16, 16)] = jnp.exp(al)

        for c in pf_copies(0, 0):
            c.start()

        @pl.loop(0, NSUP)
        def super_loop(s):
            p = lax.rem(s, 2)
            for c in pf_copies(s, p):
                c.wait()

            @pl.when(s + 1 < NSUP)
            def _():
                for c in pf_copies(s + 1, 1 - p):
                    c.start()

            copy_idx(p, 0, 0)
            g = {0: pltpu.async_copy(h_hbm.at[sidx_v.at[0]], rows_v.at[0],
                                     semg[0])}
            sc_d = {}
            for j in range(SB):
                q = j & 1
                g[j].wait()
                if j + 1 < SB:
                    if j - 1 in sc_d:
                        sc_d[j - 1].wait()
                    copy_idx(p, j + 1, 1 - q)
                    g[j + 1] = pltpu.async_copy(
                        h_hbm.at[sidx_v.at[1 - q]], rows_v.at[1 - q],
                        semg[1 - q])
                compute_ex(p, j, q)

                @plsc.parallel_loop(0, CH, step=4, unroll=2)
                def _(r):
                    for rr in range(4):
                        sc = plsc.load_gather(
                            exbuf_v, [jnp.full((16,), rr, jnp.int32) + r])
                        for c in range(D // 16):
                            rows_v[q, r + rr, pl.ds(c * 16, 16)] = (
                                rows_v[q, r + rr, pl.ds(c * 16, 16)] * sc)
                        rows_v[q, r + rr, pl.ds(D, 16)] = jnp.where(
                            lane == 0, sc, 0.0)

                sc_d[j] = pltpu.async_copy(rows_v.at[q],
                                           acc_s.at[didx_v.at[q]],
                                           semsc[q], add=True)
            sc_d[SB - 2].wait()
            sc_d[SB - 1].wait()

        plsc.subcore_barrier()
        pltpu.sync_copy(acc_s.at[pl.ds(sid * RPT, RPT)],
                        acc_hbm.at[cid, pl.ds(sid * RPT, RPT)])

    return k(h_ext, adst, we, eif, ea4)


def _attm(att_src, att_dst):
    a16 = jnp.zeros((D, 16), jnp.float32).at[:, 0].set(att_src)
    a8 = jnp.zeros((D, 8), jnp.float32).at[:, 0].set(att_dst)
    return a16, a8


def _we16(We, att_edge):
    return jnp.zeros((16,), jnp.float32).at[:DE].set(We @ att_edge)


def kernel(x, edge_index, edge_attr, W1, att_src1, att_dst1, We1, att_edge1,
           b1, W2, att_src2, att_dst2, We2, att_edge2, b2):
    eif = edge_index.reshape(2 * E)
    ea4 = edge_attr.reshape(E * DE)

    as1, ad1 = _attm(att_src1, att_dst1)
    as2, ad2 = _attm(att_src2, att_dst2)

    h1, av1 = _tc_pre(x, W1, as1, ad1)
    acc1 = _sc_layer(h1, av1[:, 0], _we16(We1, att_edge1), eif, ea4)
    h2, av2 = _tc_mid(acc1, b1.reshape(1, D), W2, as2, ad2)
    acc2 = _sc_layer(h2, av2[:, 0], _we16(We2, att_edge2), eif, ea4)
    return _tc_final(acc2, b2.reshape(1, D))


# speedup vs baseline: 1.0018x; 1.0018x over previous
import functools

import jax
import jax.numpy as jnp
from jax import lax
from jax.experimental import pallas as pl
from jax.experimental.pallas import tpu as pltpu
from jax.experimental.pallas import tpu_sc as plsc

N = 10000
E = 320000
D = 128
DE = 4

NC = 2
NS = 16
NW = NC * NS
EPW = E // NW
CH = 80
NCH = EPW // CH
AW = D + 16
RPT = N // NS

_PREC = jax.lax.Precision.HIGHEST


def _hext_store(h_ref, av_ref, h, asrc_vec, adst_vec):
    h_ref[:, :D] = h
    h_ref[:, D:] = jnp.dot(h, asrc_vec, precision=_PREC,
                           preferred_element_type=jnp.float32)
    av_ref[...] = jnp.dot(h, adst_vec, precision=_PREC,
                          preferred_element_type=jnp.float32)


def _tc_pre(x, W, As, Ad):
    R = 2000

    def body(x_ref, w_ref, as_ref, ad_ref, h_ref, av_ref):
        h = jnp.dot(x_ref[...], w_ref[...], precision=_PREC,
                    preferred_element_type=jnp.float32)
        _hext_store(h_ref, av_ref, h, as_ref[...], ad_ref[...])

    return pl.pallas_call(
        body,
        grid=(N // R,),
        in_specs=[
            pl.BlockSpec((R, D), lambda i: (i, 0)),
            pl.BlockSpec((D, D), lambda i: (0, 0)),
            pl.BlockSpec((D, 16), lambda i: (0, 0)),
            pl.BlockSpec((D, 8), lambda i: (0, 0)),
        ],
        out_specs=[
            pl.BlockSpec((R, AW), lambda i: (i, 0)),
            pl.BlockSpec((R, 8), lambda i: (i, 0)),
        ],
        out_shape=[
            jax.ShapeDtypeStruct((N, AW), jnp.float32),
            jax.ShapeDtypeStruct((N, 8), jnp.float32),
        ],
    )(x, W, As, Ad)


def _tc_mid(acc, b1, W2, As, Ad):
    R = 2000

    def body(acc_ref, b_ref, w_ref, as_ref, ad_ref, h_ref, av_ref):
        num = acc_ref[0] + acc_ref[1]
        den = num[:, D:D + 1] + 1e-16
        x2 = jnp.maximum(num[:, :D] / den + b_ref[...], 0.0)
        h2 = jnp.dot(x2, w_ref[...], precision=_PREC,
                     preferred_element_type=jnp.float32)
        _hext_store(h_ref, av_ref, h2, as_ref[...], ad_ref[...])

    return pl.pallas_call(
        body,
        grid=(N // R,),
        in_specs=[
            pl.BlockSpec((2, R, AW), lambda i: (0, i, 0)),
            pl.BlockSpec((1, D), lambda i: (0, 0)),
            pl.BlockSpec((D, D), lambda i: (0, 0)),
            pl.BlockSpec((D, 16), lambda i: (0, 0)),
            pl.BlockSpec((D, 8), lambda i: (0, 0)),
        ],
        out_specs=[
            pl.BlockSpec((R, AW), lambda i: (i, 0)),
            pl.BlockSpec((R, 8), lambda i: (i, 0)),
        ],
        out_shape=[
            jax.ShapeDtypeStruct((N, AW), jnp.float32),
            jax.ShapeDtypeStruct((N, 8), jnp.float32),
        ],
    )(acc, b1, W2, As, Ad)


def _tc_final(acc, b2):
    R = 2000

    def body(acc_ref, b_ref, o_ref):
        num = acc_ref[0] + acc_ref[1]
        den = num[:, D:D + 1] + 1e-16
        o_ref[...] = num[:, :D] / den + b_ref[...]

    return pl.pallas_call(
        body,
        grid=(N // R,),
        in_specs=[
            pl.BlockSpec((2, R, AW), lambda i: (0, i, 0)),
            pl.BlockSpec((1, D), lambda i: (0, 0)),
        ],
        out_specs=pl.BlockSpec((R, D), lambda i: (i, 0)),
        out_shape=jax.ShapeDtypeStruct((N, D), jnp.float32),
    )(acc, b2)


SB = 5
NSUP = NCH // SB
SCW = SB * CH


def _sc_layer(h_ext, adst, we, eif, ea4):
    mesh = plsc.VectorSubcoreMesh(core_axis_name="c", subcore_axis_name="s",
                                  num_cores=NC, num_subcores=NS)

    @functools.partial(
        pl.kernel,
        out_type=jax.ShapeDtypeStruct((NC, N, AW), jnp.float32),
        mesh=mesh,
        compiler_params=pltpu.CompilerParams(use_tc_tiling_on_sc=False,
                                             needs_layout_passes=False),
        scratch_types=[
            pltpu.VMEM((N,), jnp.float32),
            pltpu.VMEM((16,), jnp.float32),
            pltpu.VMEM((2, SCW), jnp.int32),
            pltpu.VMEM((2, SCW), jnp.int32),
            pltpu.VMEM((2, SCW * DE), jnp.float32),
            pltpu.VMEM((2, CH), jnp.int32),
            pltpu.VMEM((2, CH), jnp.int32),
            pltpu.VMEM((2, CH, AW), jnp.float32),
            pltpu.VMEM((CH,), jnp.float32),
            pltpu.VMEM_SHARED((N, AW), jnp.float32),
            pltpu.SemaphoreType.DMA,
            pltpu.SemaphoreType.DMA,
            pltpu.SemaphoreType.DMA,
            pltpu.SemaphoreType.DMA,
            pltpu.SemaphoreType.DMA,
        ],
    )
    def k(h_hbm, adst_hbm, we_hbm, ei_hbm, ea_hbm, acc_hbm,
          adst_v, we_v, ssrc_v, sdst_v, sea_v, sidx_v, didx_v, rows_v,
          exbuf_v, acc_s, semg0, semg1, sempf, semsc0, semsc1):
        cid = lax.axis_index("c")
        sid = lax.axis_index("s")
        wid = sid * NC + cid
        semg = (semg0, semg1)
        semsc = (semsc0, semsc1)

        pltpu.sync_copy(adst_hbm, adst_v)
        pltpu.sync_copy(we_hbm, we_v)

        zero16 = jnp.zeros((16,), jnp.float32)

        @pl.loop(0, CH)
        def _(r):
            for c in range(AW // 16):
                rows_v[0, r, pl.ds(c * 16, 16)] = zero16

        for kk in range(RPT // CH):
            pltpu.sync_copy(rows_v.at[0],
                            acc_s.at[pl.ds(sid * RPT + kk * CH, CH)])
        rem = RPT % CH
        if rem:
            pltpu.sync_copy(rows_v.at[0, pl.ds(0, rem)],
                            acc_s.at[pl.ds(sid * RPT + RPT - rem, rem)])
        plsc.subcore_barrier()

        lane = lax.iota(jnp.int32, 16)
        wv = we_v[...]
        wb = [jnp.broadcast_to(wv[j], (16,)) for j in range(DE)]

        def pf_copies(s, p):
            eoff = wid * EPW + s * SCW
            return (
                pltpu.make_async_copy(ei_hbm.at[pl.ds(eoff, SCW)],
                                      ssrc_v.at[p], sempf),
                pltpu.make_async_copy(ei_hbm.at[pl.ds(E + eoff, SCW)],
                                      sdst_v.at[p], sempf),
                pltpu.make_async_copy(ea_hbm.at[pl.ds(eoff * DE, SCW * DE)],
                                      sea_v.at[p], sempf),
            )

        def copy_idx(p, j, slot):
            for g in range(CH // 16):
                sidx_v[slot, pl.ds(g * 16, 16)] = (
                    ssrc_v[p, pl.ds(j * CH + g * 16, 16)])
                didx_v[slot, pl.ds(g * 16, 16)] = (
                    sdst_v[p, pl.ds(j * CH + g * 16, 16)])

        def compute_ex(p, j, q):
            colD = jnp.full((16,), D, jnp.int32)
            qf = jnp.full((16,), q, jnp.int32)
            pf = jnp.broadcast_to(p, (16,))
            for g in range(CH // 16):
                rowi = g * 16 + lane
                a = (plsc.load_gather(rows_v, [qf, rowi, colD])
                     + plsc.load_gather(adst_v,
                                        [didx_v[q, pl.ds(g * 16, 16)]]))
                eb = (j * CH + g * 16 + lane) * DE
                ae = wb[0] * plsc.load_gather(sea_v, [pf, eb])
                for jj in range(1, DE):
                    ae = ae + wb[jj] * plsc.load_gather(sea_v, [pf, eb + jj])
                al = a + ae
                al = jnp.maximum(al, 0.2 * al)
                exbuf_v[pl.ds(g * 16, 16)] = jnp.exp(al)

        for c in pf_copies(0, 0):
            c.start()

        @pl.loop(0, NSUP)
        def super_loop(s):
            p = lax.rem(s, 2)
            for c in pf_copies(s, p):
                c.wait()

            @pl.when(s + 1 < NSUP)
            def _():
                for c in pf_copies(s + 1, 1 - p):
                    c.start()

            copy_idx(p, 0, 0)
            g = {0: pltpu.async_copy(h_hbm.at[sidx_v.at[0]], rows_v.at[0],
                                     semg[0])}
            sc_d = {}
            for j in range(SB):
                q = j & 1
                g[j].wait()
                if j + 1 < SB:
                    if j - 1 in sc_d:
                        sc_d[j - 1].wait()
                    copy_idx(p, j + 1, 1 - q)
                    g[j + 1] = pltpu.async_copy(
                        h_hbm.at[sidx_v.at[1 - q]], rows_v.at[1 - q],
                        semg[1 - q])
                compute_ex(p, j, q)

                @plsc.parallel_loop(0, CH, step=2, unroll=2)
                def _(r):
                    for rr in range(2):
                        sc = plsc.load_gather(
                            exbuf_v, [jnp.full((16,), rr, jnp.int32) + r])
                        for c in range(D // 16):
                            rows_v[q, r + rr, pl.ds(c * 16, 16)] = (
                                rows_v[q, r + rr, pl.ds(c * 16, 16)] * sc)
                        rows_v[q, r + rr, pl.ds(D, 16)] = jnp.where(
                            lane == 0, sc, 0.0)

                sc_d[j] = pltpu.async_copy(rows_v.at[q],
                                           acc_s.at[didx_v.at[q]],
                                           semsc[q], add=True)
            sc_d[SB - 2].wait()
            sc_d[SB - 1].wait()

        plsc.subcore_barrier()
        pltpu.sync_copy(acc_s.at[pl.ds(sid * RPT, RPT)],
                        acc_hbm.at[cid, pl.ds(sid * RPT, RPT)])

    return k(h_ext, adst, we, eif, ea4)


def _attm(att_src, att_dst):
    a16 = jnp.zeros((D, 16), jnp.float32).at[:, 0].set(att_src)
    a8 = jnp.zeros((D, 8), jnp.float32).at[:, 0].set(att_dst)
    return a16, a8


def _we16(We, att_edge):
    return jnp.zeros((16,), jnp.float32).at[:DE].set(We @ att_edge)


def kernel(x, edge_index, edge_attr, W1, att_src1, att_dst1, We1, att_edge1,
           b1, W2, att_src2, att_dst2, We2, att_edge2, b2):
    eif = edge_index.reshape(2 * E)
    ea4 = edge_attr.reshape(E * DE)

    as1, ad1 = _attm(att_src1, att_dst1)
    as2, ad2 = _attm(att_src2, att_dst2)

    h1, av1 = _tc_pre(x, W1, as1, ad1)
    acc1 = _sc_layer(h1, av1[:, 0], _we16(We1, att_edge1), eif, ea4)
    h2, av2 = _tc_mid(acc1, b1.reshape(1, D), W2, as2, ad2)
    acc2 = _sc_layer(h2, av2[:, 0], _we16(We2, att_edge2), eif, ea4)
    return _tc_final(acc2, b2.reshape(1, D))
